# Initial kernel scaffold; baseline (speedup 1.0000x reference)
import jax
import jax.numpy as jnp
from jax.experimental import pallas as pl


def kernel(x, edge_index, W1, W2, W3):
    raise NotImplementedError("write your pallas kernel here")


# speedup vs baseline: 3.4266x; 3.4266x over previous
import functools

import jax
import jax.numpy as jnp
from jax import lax
from jax.experimental import pallas as pl
from jax.experimental.pallas import tpu as pltpu
from jax.experimental.pallas import tpu_sc as plsc

NC = 2
NS = 16
NW = NC * NS
LANES = 16
CHUNK = 80
DW = 128


def _fill2d(ref, nrows, ncols, value):
    vec = jnp.full((LANES,), value, jnp.float32)

    def row(i, carry):
        for j in range(ncols // LANES):
            ref[i, pl.ds(j * LANES, LANES)] = vec
        return carry

    lax.fori_loop(0, nrows, row, 0)


def _sc_mesh():
    return plsc.VectorSubcoreMesh(core_axis_name="c", subcore_axis_name="s")


@functools.lru_cache(maxsize=None)
def _build_agg(e, n_pad):
    ept = e // NW
    nchunks = ept // CHUNK
    rpt = n_pad // NS
    assert ept % CHUNK == 0 and rpt % CHUNK == 0

    def body(h_hbm, src_hbm, dst_hbm, part_hbm, idx_s, idx_d, rows_v, acc_sh,
             sem):
        c = lax.axis_index("c")
        s = lax.axis_index("s")
        wid = c * NS + s
        row0 = s * rpt

        _fill2d(rows_v, CHUNK, DW, 0.0)
        for t in range(rpt // CHUNK):
            pltpu.sync_copy(rows_v, acc_sh.at[pl.ds(row0 + t * CHUNK, CHUNK)])
        plsc.subcore_barrier()

        def chunk(g, carry):
            e0 = pl.multiple_of(wid * ept + g * CHUNK, 8)
            pltpu.sync_copy(src_hbm.at[pl.ds(e0, CHUNK)], idx_s)
            pltpu.async_copy(h_hbm.at[idx_s], rows_v, sem).wait()
            pltpu.sync_copy(dst_hbm.at[pl.ds(e0, CHUNK)], idx_d)
            pltpu.sync_copy(rows_v, acc_sh.at[idx_d], add=True)
            return carry

        lax.fori_loop(0, nchunks, chunk, 0)
        plsc.subcore_barrier()

        pltpu.sync_copy(acc_sh.at[pl.ds(row0, rpt)],
                        part_hbm.at[c, pl.ds(row0, rpt)])

    return pl.kernel(
        body,
        out_type=jax.ShapeDtypeStruct((NC, n_pad, DW), jnp.float32),
        mesh=_sc_mesh(),
        scratch_types=[
            pltpu.VMEM((CHUNK,), jnp.int32),
            pltpu.VMEM((CHUNK,), jnp.int32),
            pltpu.VMEM((CHUNK, DW), jnp.float32),
            pltpu.VMEM_SHARED((n_pad, DW), jnp.float32),
            pltpu.SemaphoreType.DMA,
        ],
    )


@functools.lru_cache(maxsize=None)
def _build_deg(e, n_pad):
    ept = e // NW
    nchunks = ept // CHUNK
    rpt = n_pad // NS
    assert rpt % CHUNK == 0

    def body(dst_hbm, degp_hbm, idx_d, ones_v, zb_v, deg_sh, sem):
        c = lax.axis_index("c")
        s = lax.axis_index("s")
        wid = c * NS + s
        row0 = s * rpt

        _fill2d(ones_v, CHUNK, DW, 1.0)
        _fill2d(zb_v, CHUNK, DW, 0.0)
        for t in range(rpt // CHUNK):
            pltpu.sync_copy(zb_v, deg_sh.at[pl.ds(row0 + t * CHUNK, CHUNK)])
        plsc.subcore_barrier()

        def chunk(g, carry):
            e0 = pl.multiple_of(wid * ept + g * CHUNK, 8)
            pltpu.sync_copy(dst_hbm.at[pl.ds(e0, CHUNK)], idx_d)
            pltpu.sync_copy(ones_v, deg_sh.at[idx_d], add=True)
            return carry

        lax.fori_loop(0, nchunks, chunk, 0)
        plsc.subcore_barrier()
        pltpu.sync_copy(deg_sh.at[pl.ds(row0, rpt)],
                        degp_hbm.at[c, pl.ds(row0, rpt)])

    return pl.kernel(
        body,
        out_type=jax.ShapeDtypeStruct((NC, n_pad, DW), jnp.float32),
        mesh=_sc_mesh(),
        scratch_types=[
            pltpu.VMEM((CHUNK,), jnp.int32),
            pltpu.VMEM((CHUNK, DW), jnp.float32),
            pltpu.VMEM((CHUNK, DW), jnp.float32),
            pltpu.VMEM_SHARED((n_pad, DW), jnp.float32),
            pltpu.SemaphoreType.DMA,
        ],
    )


@functools.lru_cache(maxsize=None)
def _build_tc1(n_pad, h2, r_blk=512):
    def body(p_ref, dp_ref, w_ref, ha_ref, hb_ref, r_ref):
        d = dp_ref[0, :, 0:1] + dp_ref[1, :, 0:1]
        r = 1.0 / jnp.maximum(d, 1.0)
        p = p_ref[0] + p_ref[1]
        hh = jnp.dot(p, w_ref[...], preferred_element_type=jnp.float32)
        hh = jnp.maximum(hh * r, 0.0)
        ha_ref[...] = hh[:, :DW]
        hb_ref[...] = hh[:, DW:]
        r_ref[...] = r

    return pl.pallas_call(
        body,
        grid=(n_pad // r_blk,),
        in_specs=[
            pl.BlockSpec((NC, r_blk, DW), lambda i: (0, i, 0)),
            pl.BlockSpec((NC, r_blk, DW), lambda i: (0, i, 0)),
            pl.BlockSpec((DW, h2), lambda i: (0, 0)),
        ],
        out_specs=[
            pl.BlockSpec((r_blk, DW), lambda i: (i, 0)),
            pl.BlockSpec((r_blk, DW), lambda i: (i, 0)),
            pl.BlockSpec((r_blk, 1), lambda i: (i, 0)),
        ],
        out_shape=[
            jax.ShapeDtypeStruct((n_pad, DW), jnp.float32),
            jax.ShapeDtypeStruct((n_pad, DW), jnp.float32),
            jax.ShapeDtypeStruct((n_pad, 1), jnp.float32),
        ],
    )


@functools.lru_cache(maxsize=None)
def _build_tc2(n_pad, h2, r_blk=512):
    def body(pa_ref, pb_ref, r_ref, w_ref, ha_ref, hb_ref):
        p = jnp.concatenate([pa_ref[0] + pa_ref[1], pb_ref[0] + pb_ref[1]],
                            axis=1)
        hh = jnp.dot(p, w_ref[...], preferred_element_type=jnp.float32)
        hh = jnp.maximum(hh * r_ref[...], 0.0)
        ha_ref[...] = hh[:, :DW]
        hb_ref[...] = hh[:, DW:]

    return pl.pallas_call(
        body,
        grid=(n_pad // r_blk,),
        in_specs=[
            pl.BlockSpec((NC, r_blk, DW), lambda i: (0, i, 0)),
            pl.BlockSpec((NC, r_blk, DW), lambda i: (0, i, 0)),
            pl.BlockSpec((r_blk, 1), lambda i: (i, 0)),
            pl.BlockSpec((h2, h2), lambda i: (0, 0)),
        ],
        out_specs=[
            pl.BlockSpec((r_blk, DW), lambda i: (i, 0)),
            pl.BlockSpec((r_blk, DW), lambda i: (i, 0)),
        ],
        out_shape=[
            jax.ShapeDtypeStruct((n_pad, DW), jnp.float32),
            jax.ShapeDtypeStruct((n_pad, DW), jnp.float32),
        ],
    )


@functools.lru_cache(maxsize=None)
def _build_tc3(n_pad, h2, h_dim, r_blk=512):
    def body(pa_ref, pb_ref, r_ref, w_ref, h_ref):
        p = jnp.concatenate([pa_ref[0] + pa_ref[1], pb_ref[0] + pb_ref[1]],
                            axis=1)
        hh = jnp.dot(p, w_ref[...], preferred_element_type=jnp.float32)
        hh = jnp.maximum(hh * r_ref[...], 0.0)
        h_ref[...] = jnp.minimum(hh, 1000.0)

    return pl.pallas_call(
        body,
        grid=(n_pad // r_blk,),
        in_specs=[
            pl.BlockSpec((NC, r_blk, DW), lambda i: (0, i, 0)),
            pl.BlockSpec((NC, r_blk, DW), lambda i: (0, i, 0)),
            pl.BlockSpec((r_blk, 1), lambda i: (i, 0)),
            pl.BlockSpec((h2, h_dim), lambda i: (0, 0)),
        ],
        out_specs=pl.BlockSpec((r_blk, h_dim), lambda i: (i, 0)),
        out_shape=jax.ShapeDtypeStruct((n_pad, h_dim), jnp.float32),
    )


def kernel(x, edge_index, W1, W2, W3):
    n, d = x.shape
    h_dim = W1.shape[1]
    e = edge_index.shape[1]
    assert d == DW and h_dim == 192
    h2 = 2 * DW

    src = edge_index[0].astype(jnp.int32)
    dst = edge_index[1].astype(jnp.int32)

    align = NS * 640
    n_pad = -(-n // align) * align

    W1p = jnp.pad(W1, ((0, 0), (0, h2 - h_dim)))
    W2p = jnp.pad(W2, ((0, h2 - h_dim), (0, h2 - h_dim)))
    W3p = jnp.pad(W3, ((0, h2 - h_dim), (0, 0)))

    agg = _build_agg(e, n_pad)

    degp = _build_deg(e, n_pad)(dst)
    part1 = agg(x, src, dst)
    h1a, h1b, rdeg = _build_tc1(n_pad, h2)(part1, degp, W1p)
    p2a = agg(h1a, src, dst)
    p2b = agg(h1b, src, dst)
    h2a, h2b = _build_tc2(n_pad, h2)(p2a, p2b, rdeg, W2p)
    p3a = agg(h2a, src, dst)
    p3b = agg(h2b, src, dst)
    h3 = _build_tc3(n_pad, h2, h_dim)(p3a, p3b, rdeg, W3p)
    return h3[:n]


# speedup vs baseline: 5.3317x; 1.5560x over previous
import functools

import jax
import jax.numpy as jnp
from jax import lax
from jax.experimental import pallas as pl
from jax.experimental.pallas import tpu as pltpu
from jax.experimental.pallas import tpu_sc as plsc

NC = 2
NS = 16
NW = NC * NS
LANES = 16
CHUNK = 80
DW = 128


def _fill2d(ref, nrows, ncols, value):
    vec = jnp.full((LANES,), value, jnp.float32)

    def row(i, carry):
        for j in range(ncols // LANES):
            ref[i, pl.ds(j * LANES, LANES)] = vec
        return carry

    lax.fori_loop(0, nrows, row, 0)


def _sc_mesh():
    return plsc.VectorSubcoreMesh(core_axis_name="c", subcore_axis_name="s")


@functools.lru_cache(maxsize=None)
def _build_agg(e, n_pad):
    ept = e // NW
    nchunks = ept // CHUNK
    rpt = n_pad // NS
    assert ept % CHUNK == 0 and rpt % CHUNK == 0

    def body(h_hbm, src_hbm, dst_hbm, part_hbm, idx_s0, idx_s1, idx_d,
             rows0, rows1, acc_sh, sem0, sem1):
        c = lax.axis_index("c")
        s = lax.axis_index("s")
        wid = c * NS + s
        row0 = s * rpt
        idx_s = (idx_s0, idx_s1)
        rows = (rows0, rows1)
        sems = (sem0, sem1)

        _fill2d(rows0, CHUNK, DW, 0.0)
        for t in range(rpt // CHUNK):
            pltpu.sync_copy(rows0, acc_sh.at[pl.ds(row0 + t * CHUNK, CHUNK)])
        plsc.subcore_barrier()

        def e_at(g):
            return pl.multiple_of(wid * ept + g * CHUNK, 8)

        def step(g, b, prefetch_next):
            if prefetch_next:
                b2 = 1 - b
                pltpu.sync_copy(src_hbm.at[pl.ds(e_at(g + 1), CHUNK)],
                                idx_s[b2])
                pltpu.async_copy(h_hbm.at[idx_s[b2]], rows[b2], sems[b2])
            pltpu.sync_copy(dst_hbm.at[pl.ds(e_at(g), CHUNK)], idx_d)
            pltpu.make_async_copy(h_hbm.at[idx_s[b]], rows[b], sems[b]).wait()
            pltpu.sync_copy(rows[b], acc_sh.at[idx_d], add=True)

        pltpu.sync_copy(src_hbm.at[pl.ds(e_at(0), CHUNK)], idx_s0)
        pltpu.async_copy(h_hbm.at[idx_s0], rows0, sem0)

        def pair(i, carry):
            g = 2 * i
            step(g, 0, True)
            step(g + 1, 1, True)
            return carry

        lax.fori_loop(0, (nchunks - 1) // 2, pair, 0)
        if (nchunks - 1) % 2 == 1:
            step(nchunks - 2, (nchunks - 2) % 2, True)
        step(nchunks - 1, (nchunks - 1) % 2, False)

        plsc.subcore_barrier()

        pltpu.sync_copy(acc_sh.at[pl.ds(row0, rpt)],
                        part_hbm.at[c, pl.ds(row0, rpt)])

    return pl.kernel(
        body,
        out_type=jax.ShapeDtypeStruct((NC, n_pad, DW), jnp.float32),
        mesh=_sc_mesh(),
        scratch_types=[
            pltpu.VMEM((CHUNK,), jnp.int32),
            pltpu.VMEM((CHUNK,), jnp.int32),
            pltpu.VMEM((CHUNK,), jnp.int32),
            pltpu.VMEM((CHUNK, DW), jnp.float32),
            pltpu.VMEM((CHUNK, DW), jnp.float32),
            pltpu.VMEM_SHARED((n_pad, DW), jnp.float32),
            pltpu.SemaphoreType.DMA,
            pltpu.SemaphoreType.DMA,
        ],
    )


@functools.lru_cache(maxsize=None)
def _build_deg(e, n_pad):
    ept = e // NW
    nchunks = ept // CHUNK
    rpt = n_pad // NS
    assert rpt % CHUNK == 0

    def body(dst_hbm, degp_hbm, idx_d, ones_v, zb_v, deg_sh, sem):
        c = lax.axis_index("c")
        s = lax.axis_index("s")
        wid = c * NS + s
        row0 = s * rpt

        _fill2d(ones_v, CHUNK, DW, 1.0)
        _fill2d(zb_v, CHUNK, DW, 0.0)
        for t in range(rpt // CHUNK):
            pltpu.sync_copy(zb_v, deg_sh.at[pl.ds(row0 + t * CHUNK, CHUNK)])
        plsc.subcore_barrier()

        def chunk(g, carry):
            e0 = pl.multiple_of(wid * ept + g * CHUNK, 8)
            pltpu.sync_copy(dst_hbm.at[pl.ds(e0, CHUNK)], idx_d)
            pltpu.sync_copy(ones_v, deg_sh.at[idx_d], add=True)
            return carry

        lax.fori_loop(0, nchunks, chunk, 0)
        plsc.subcore_barrier()
        pltpu.sync_copy(deg_sh.at[pl.ds(row0, rpt)],
                        degp_hbm.at[c, pl.ds(row0, rpt)])

    return pl.kernel(
        body,
        out_type=jax.ShapeDtypeStruct((NC, n_pad, DW), jnp.float32),
        mesh=_sc_mesh(),
        scratch_types=[
            pltpu.VMEM((CHUNK,), jnp.int32),
            pltpu.VMEM((CHUNK, DW), jnp.float32),
            pltpu.VMEM((CHUNK, DW), jnp.float32),
            pltpu.VMEM_SHARED((n_pad, DW), jnp.float32),
            pltpu.SemaphoreType.DMA,
        ],
    )


@functools.lru_cache(maxsize=None)
def _build_tc1(n_pad, h2, r_blk=512):
    def body(p_ref, dp_ref, w_ref, ha_ref, hb_ref, r_ref):
        d = dp_ref[0, :, 0:1] + dp_ref[1, :, 0:1]
        r = 1.0 / jnp.maximum(d, 1.0)
        p = p_ref[0] + p_ref[1]
        hh = jnp.dot(p, w_ref[...], preferred_element_type=jnp.float32)
        hh = jnp.maximum(hh * r, 0.0)
        ha_ref[...] = hh[:, :DW]
        hb_ref[...] = hh[:, DW:]
        r_ref[...] = r

    return pl.pallas_call(
        body,
        grid=(n_pad // r_blk,),
        in_specs=[
            pl.BlockSpec((NC, r_blk, DW), lambda i: (0, i, 0)),
            pl.BlockSpec((NC, r_blk, DW), lambda i: (0, i, 0)),
            pl.BlockSpec((DW, h2), lambda i: (0, 0)),
        ],
        out_specs=[
            pl.BlockSpec((r_blk, DW), lambda i: (i, 0)),
            pl.BlockSpec((r_blk, DW), lambda i: (i, 0)),
            pl.BlockSpec((r_blk, 1), lambda i: (i, 0)),
        ],
        out_shape=[
            jax.ShapeDtypeStruct((n_pad, DW), jnp.float32),
            jax.ShapeDtypeStruct((n_pad, DW), jnp.float32),
            jax.ShapeDtypeStruct((n_pad, 1), jnp.float32),
        ],
    )


@functools.lru_cache(maxsize=None)
def _build_tc2(n_pad, h2, r_blk=512):
    def body(pa_ref, pb_ref, r_ref, w_ref, ha_ref, hb_ref):
        p = jnp.concatenate([pa_ref[0] + pa_ref[1], pb_ref[0] + pb_ref[1]],
                            axis=1)
        hh = jnp.dot(p, w_ref[...], preferred_element_type=jnp.float32)
        hh = jnp.maximum(hh * r_ref[...], 0.0)
        ha_ref[...] = hh[:, :DW]
        hb_ref[...] = hh[:, DW:]

    return pl.pallas_call(
        body,
        grid=(n_pad // r_blk,),
        in_specs=[
            pl.BlockSpec((NC, r_blk, DW), lambda i: (0, i, 0)),
            pl.BlockSpec((NC, r_blk, DW), lambda i: (0, i, 0)),
            pl.BlockSpec((r_blk, 1), lambda i: (i, 0)),
            pl.BlockSpec((h2, h2), lambda i: (0, 0)),
        ],
        out_specs=[
            pl.BlockSpec((r_blk, DW), lambda i: (i, 0)),
            pl.BlockSpec((r_blk, DW), lambda i: (i, 0)),
        ],
        out_shape=[
            jax.ShapeDtypeStruct((n_pad, DW), jnp.float32),
            jax.ShapeDtypeStruct((n_pad, DW), jnp.float32),
        ],
    )


@functools.lru_cache(maxsize=None)
def _build_tc3(n_pad, h2, h_dim, r_blk=512):
    def body(pa_ref, pb_ref, r_ref, w_ref, h_ref):
        p = jnp.concatenate([pa_ref[0] + pa_ref[1], pb_ref[0] + pb_ref[1]],
                            axis=1)
        hh = jnp.dot(p, w_ref[...], preferred_element_type=jnp.float32)
        hh = jnp.maximum(hh * r_ref[...], 0.0)
        h_ref[...] = jnp.minimum(hh, 1000.0)

    return pl.pallas_call(
        body,
        grid=(n_pad // r_blk,),
        in_specs=[
            pl.BlockSpec((NC, r_blk, DW), lambda i: (0, i, 0)),
            pl.BlockSpec((NC, r_blk, DW), lambda i: (0, i, 0)),
            pl.BlockSpec((r_blk, 1), lambda i: (i, 0)),
            pl.BlockSpec((h2, h_dim), lambda i: (0, 0)),
        ],
        out_specs=pl.BlockSpec((r_blk, h_dim), lambda i: (i, 0)),
        out_shape=jax.ShapeDtypeStruct((n_pad, h_dim), jnp.float32),
    )


def kernel(x, edge_index, W1, W2, W3):
    n, d = x.shape
    h_dim = W1.shape[1]
    e = edge_index.shape[1]
    assert d == DW and h_dim == 192
    h2 = 2 * DW

    src = edge_index[0].astype(jnp.int32)
    dst = edge_index[1].astype(jnp.int32)

    align = NS * 640
    n_pad = -(-n // align) * align

    W1p = jnp.pad(W1, ((0, 0), (0, h2 - h_dim)))
    W2p = jnp.pad(W2, ((0, h2 - h_dim), (0, h2 - h_dim)))
    W3p = jnp.pad(W3, ((0, h2 - h_dim), (0, 0)))

    agg = _build_agg(e, n_pad)

    degp = _build_deg(e, n_pad)(dst)
    part1 = agg(x, src, dst)
    h1a, h1b, rdeg = _build_tc1(n_pad, h2)(part1, degp, W1p)
    p2a = agg(h1a, src, dst)
    p2b = agg(h1b, src, dst)
    h2a, h2b = _build_tc2(n_pad, h2)(p2a, p2b, rdeg, W2p)
    p3a = agg(h2a, src, dst)
    p3b = agg(h2b, src, dst)
    h3 = _build_tc3(n_pad, h2, h_dim)(p3a, p3b, rdeg, W3p)
    return h3[:n]


# speedup vs baseline: 7.0378x; 1.3200x over previous
import functools

import jax
import jax.numpy as jnp
from jax import lax
from jax.experimental import pallas as pl
from jax.experimental.pallas import tpu as pltpu
from jax.experimental.pallas import tpu_sc as plsc

NC = 2
NS = 16
NW = NC * NS
LANES = 16
CHUNK = 80
DW = 128


def _fill2d(ref, nrows, ncols, value):
    vec = jnp.full((LANES,), value, jnp.float32)

    def row(i, carry):
        for j in range(ncols // LANES):
            ref[i, pl.ds(j * LANES, LANES)] = vec
        return carry

    lax.fori_loop(0, nrows, row, 0)


def _sc_mesh():
    return plsc.VectorSubcoreMesh(core_axis_name="c", subcore_axis_name="s")


@functools.lru_cache(maxsize=None)
def _build_agg(e, n_pad):
    ept = e // NW
    nchunks = ept // CHUNK
    rpt = n_pad // NS
    assert ept % CHUNK == 0 and rpt % CHUNK == 0

    assert nchunks % 2 == 1 and nchunks >= 5

    def body(h_hbm, src_hbm, dst_hbm, part_hbm, idx_s0, idx_s1, idx_d0,
             idx_d1, rows0, rows1, acc_sh, sg0, sg1, ss0, ss1):
        c = lax.axis_index("c")
        s = lax.axis_index("s")
        wid = c * NS + s
        row0 = s * rpt
        idx_s = (idx_s0, idx_s1)
        idx_d = (idx_d0, idx_d1)
        rows = (rows0, rows1)
        sg = (sg0, sg1)
        ss = (ss0, ss1)

        _fill2d(rows0, CHUNK, DW, 0.0)
        for t in range(rpt // CHUNK):
            pltpu.sync_copy(rows0, acc_sh.at[pl.ds(row0 + t * CHUNK, CHUNK)])
        plsc.subcore_barrier()

        def e_at(g):
            return pl.multiple_of(wid * ept + g * CHUNK, 8)

        def wait_scatter(b):
            pltpu.make_async_copy(rows[b], acc_sh.at[idx_d[b]], ss[b]).wait()

        def step(g, b, prefetch, drain):
            b2 = 1 - b
            if prefetch:
                pltpu.sync_copy(src_hbm.at[pl.ds(e_at(g + 1), CHUNK)],
                                idx_s[b2])
            if drain:
                wait_scatter(b2)
            if prefetch:
                pltpu.async_copy(h_hbm.at[idx_s[b2]], rows[b2], sg[b2])
            pltpu.sync_copy(dst_hbm.at[pl.ds(e_at(g), CHUNK)], idx_d[b])
            pltpu.make_async_copy(h_hbm.at[idx_s[b]], rows[b], sg[b]).wait()
            pltpu.async_copy(rows[b], acc_sh.at[idx_d[b]], ss[b], add=True)

        pltpu.sync_copy(src_hbm.at[pl.ds(e_at(0), CHUNK)], idx_s0)
        pltpu.async_copy(h_hbm.at[idx_s0], rows0, sg0)
        step(0, 0, True, False)
        step(1, 1, True, True)

        def pair(i, carry):
            g = 2 * i
            step(g, 0, True, True)
            step(g + 1, 1, True, True)
            return carry

        lax.fori_loop(1, (nchunks - 1) // 2, pair, 0)
        step(nchunks - 1, 0, False, True)
        wait_scatter(0)

        plsc.subcore_barrier()

        pltpu.sync_copy(acc_sh.at[pl.ds(row0, rpt)],
                        part_hbm.at[c, pl.ds(row0, rpt)])

    return pl.kernel(
        body,
        out_type=jax.ShapeDtypeStruct((NC, n_pad, DW), jnp.float32),
        mesh=_sc_mesh(),
        scratch_types=[
            pltpu.VMEM((CHUNK,), jnp.int32),
            pltpu.VMEM((CHUNK,), jnp.int32),
            pltpu.VMEM((CHUNK,), jnp.int32),
            pltpu.VMEM((CHUNK,), jnp.int32),
            pltpu.VMEM((CHUNK, DW), jnp.float32),
            pltpu.VMEM((CHUNK, DW), jnp.float32),
            pltpu.VMEM_SHARED((n_pad, DW), jnp.float32),
            pltpu.SemaphoreType.DMA,
            pltpu.SemaphoreType.DMA,
            pltpu.SemaphoreType.DMA,
            pltpu.SemaphoreType.DMA,
        ],
    )


@functools.lru_cache(maxsize=None)
def _build_deg(e, n_pad):
    ept = e // NW
    nchunks = ept // CHUNK
    rpt = n_pad // NS
    assert rpt % CHUNK == 0

    assert nchunks % 2 == 1 and nchunks >= 5

    def body(dst_hbm, degp_hbm, idx_d0, idx_d1, ones_v, deg_sh, ss0, ss1):
        c = lax.axis_index("c")
        s = lax.axis_index("s")
        wid = c * NS + s
        row0 = s * rpt
        idx_d = (idx_d0, idx_d1)
        ss = (ss0, ss1)

        _fill2d(ones_v, CHUNK, DW, 0.0)
        for t in range(rpt // CHUNK):
            pltpu.sync_copy(ones_v, deg_sh.at[pl.ds(row0 + t * CHUNK, CHUNK)])
        _fill2d(ones_v, CHUNK, DW, 1.0)
        plsc.subcore_barrier()

        def step(g, b, drain):
            if drain:
                pltpu.make_async_copy(ones_v, deg_sh.at[idx_d[b]],
                                      ss[b]).wait()
            e0 = pl.multiple_of(wid * ept + g * CHUNK, 8)
            pltpu.sync_copy(dst_hbm.at[pl.ds(e0, CHUNK)], idx_d[b])
            pltpu.async_copy(ones_v, deg_sh.at[idx_d[b]], ss[b], add=True)

        step(0, 0, False)
        step(1, 1, False)

        def pair(i, carry):
            g = 2 * i
            step(g, 0, True)
            step(g + 1, 1, True)
            return carry

        lax.fori_loop(1, (nchunks - 1) // 2, pair, 0)
        step(nchunks - 1, 0, True)
        pltpu.make_async_copy(ones_v, deg_sh.at[idx_d[0]], ss[0]).wait()
        pltpu.make_async_copy(ones_v, deg_sh.at[idx_d[1]], ss[1]).wait()

        plsc.subcore_barrier()
        pltpu.sync_copy(deg_sh.at[pl.ds(row0, rpt)],
                        degp_hbm.at[c, pl.ds(row0, rpt)])

    return pl.kernel(
        body,
        out_type=jax.ShapeDtypeStruct((NC, n_pad, DW), jnp.float32),
        mesh=_sc_mesh(),
        scratch_types=[
            pltpu.VMEM((CHUNK,), jnp.int32),
            pltpu.VMEM((CHUNK,), jnp.int32),
            pltpu.VMEM((CHUNK, DW), jnp.float32),
            pltpu.VMEM_SHARED((n_pad, DW), jnp.float32),
            pltpu.SemaphoreType.DMA,
            pltpu.SemaphoreType.DMA,
        ],
    )


@functools.lru_cache(maxsize=None)
def _build_tc1(n_pad, h2, r_blk=512):
    def body(p_ref, dp_ref, w_ref, ha_ref, hb_ref, r_ref):
        d = dp_ref[0, :, 0:1] + dp_ref[1, :, 0:1]
        r = 1.0 / jnp.maximum(d, 1.0)
        p = p_ref[0] + p_ref[1]
        hh = jnp.dot(p, w_ref[...], preferred_element_type=jnp.float32)
        hh = jnp.maximum(hh * r, 0.0)
        ha_ref[...] = hh[:, :DW]
        hb_ref[...] = hh[:, DW:]
        r_ref[...] = r

    return pl.pallas_call(
        body,
        grid=(n_pad // r_blk,),
        in_specs=[
            pl.BlockSpec((NC, r_blk, DW), lambda i: (0, i, 0)),
            pl.BlockSpec((NC, r_blk, DW), lambda i: (0, i, 0)),
            pl.BlockSpec((DW, h2), lambda i: (0, 0)),
        ],
        out_specs=[
            pl.BlockSpec((r_blk, DW), lambda i: (i, 0)),
            pl.BlockSpec((r_blk, DW), lambda i: (i, 0)),
            pl.BlockSpec((r_blk, 1), lambda i: (i, 0)),
        ],
        out_shape=[
            jax.ShapeDtypeStruct((n_pad, DW), jnp.float32),
            jax.ShapeDtypeStruct((n_pad, DW), jnp.float32),
            jax.ShapeDtypeStruct((n_pad, 1), jnp.float32),
        ],
    )


@functools.lru_cache(maxsize=None)
def _build_tc2(n_pad, h2, r_blk=512):
    def body(pa_ref, pb_ref, r_ref, w_ref, ha_ref, hb_ref):
        p = jnp.concatenate([pa_ref[0] + pa_ref[1], pb_ref[0] + pb_ref[1]],
                            axis=1)
        hh = jnp.dot(p, w_ref[...], preferred_element_type=jnp.float32)
        hh = jnp.maximum(hh * r_ref[...], 0.0)
        ha_ref[...] = hh[:, :DW]
        hb_ref[...] = hh[:, DW:]

    return pl.pallas_call(
        body,
        grid=(n_pad // r_blk,),
        in_specs=[
            pl.BlockSpec((NC, r_blk, DW), lambda i: (0, i, 0)),
            pl.BlockSpec((NC, r_blk, DW), lambda i: (0, i, 0)),
            pl.BlockSpec((r_blk, 1), lambda i: (i, 0)),
            pl.BlockSpec((h2, h2), lambda i: (0, 0)),
        ],
        out_specs=[
            pl.BlockSpec((r_blk, DW), lambda i: (i, 0)),
            pl.BlockSpec((r_blk, DW), lambda i: (i, 0)),
        ],
        out_shape=[
            jax.ShapeDtypeStruct((n_pad, DW), jnp.float32),
            jax.ShapeDtypeStruct((n_pad, DW), jnp.float32),
        ],
    )


@functools.lru_cache(maxsize=None)
def _build_tc3(n_pad, h2, h_dim, r_blk=512):
    def body(pa_ref, pb_ref, r_ref, w_ref, h_ref):
        p = jnp.concatenate([pa_ref[0] + pa_ref[1], pb_ref[0] + pb_ref[1]],
                            axis=1)
        hh = jnp.dot(p, w_ref[...], preferred_element_type=jnp.float32)
        hh = jnp.maximum(hh * r_ref[...], 0.0)
        h_ref[...] = jnp.minimum(hh, 1000.0)

    return pl.pallas_call(
        body,
        grid=(n_pad // r_blk,),
        in_specs=[
            pl.BlockSpec((NC, r_blk, DW), lambda i: (0, i, 0)),
            pl.BlockSpec((NC, r_blk, DW), lambda i: (0, i, 0)),
            pl.BlockSpec((r_blk, 1), lambda i: (i, 0)),
            pl.BlockSpec((h2, h_dim), lambda i: (0, 0)),
        ],
        out_specs=pl.BlockSpec((r_blk, h_dim), lambda i: (i, 0)),
        out_shape=jax.ShapeDtypeStruct((n_pad, h_dim), jnp.float32),
    )


def kernel(x, edge_index, W1, W2, W3):
    n, d = x.shape
    h_dim = W1.shape[1]
    e = edge_index.shape[1]
    assert d == DW and h_dim == 192
    h2 = 2 * DW

    src = edge_index[0].astype(jnp.int32)
    dst = edge_index[1].astype(jnp.int32)

    align = NS * 640
    n_pad = -(-n // align) * align

    W1p = jnp.pad(W1, ((0, 0), (0, h2 - h_dim)))
    W2p = jnp.pad(W2, ((0, h2 - h_dim), (0, h2 - h_dim)))
    W3p = jnp.pad(W3, ((0, h2 - h_dim), (0, 0)))

    agg = _build_agg(e, n_pad)

    degp = _build_deg(e, n_pad)(dst)
    part1 = agg(x, src, dst)
    h1a, h1b, rdeg = _build_tc1(n_pad, h2)(part1, degp, W1p)
    p2a = agg(h1a, src, dst)
    p2b = agg(h1b, src, dst)
    h2a, h2b = _build_tc2(n_pad, h2)(p2a, p2b, rdeg, W2p)
    p3a = agg(h2a, src, dst)
    p3b = agg(h2b, src, dst)
    h3 = _build_tc3(n_pad, h2, h_dim)(p3a, p3b, rdeg, W3p)
    return h3[:n]


# speedup vs baseline: 7.8763x; 1.1191x over previous
import functools

import jax
import jax.numpy as jnp
from jax import lax
from jax.experimental import pallas as pl
from jax.experimental.pallas import tpu as pltpu
from jax.experimental.pallas import tpu_sc as plsc

NC = 2
NS = 16
NW = NC * NS
LANES = 16
CHUNK = 80
DW = 128


def _fill2d(ref, nrows, ncols, value):
    vec = jnp.full((LANES,), value, jnp.float32)

    def row(i, carry):
        for j in range(ncols // LANES):
            ref[i, pl.ds(j * LANES, LANES)] = vec
        return carry

    lax.fori_loop(0, nrows, row, 0)


def _sc_mesh():
    return plsc.VectorSubcoreMesh(core_axis_name="c", subcore_axis_name="s")


@functools.lru_cache(maxsize=None)
def _build_agg(e, n_pad):
    ept = e // NW
    nchunks = ept // CHUNK
    rpt = n_pad // NS
    assert ept % CHUNK == 0 and rpt % CHUNK == 0

    assert nchunks % 2 == 1 and nchunks >= 5

    def body(h_hbm, src_hbm, dst_hbm, part_hbm, idx_s, idx_d, rows0, rows1,
             acc_sh, si, sg0, sg1, ss0, ss1):
        c = lax.axis_index("c")
        s = lax.axis_index("s")
        wid = c * NS + s
        row0 = s * rpt
        rows = (rows0, rows1)
        sg = (sg0, sg1)
        ss = (ss0, ss1)

        pltpu.async_copy(src_hbm.at[pl.ds(wid * ept, ept)], idx_s, si)
        pltpu.async_copy(dst_hbm.at[pl.ds(wid * ept, ept)], idx_d, si)
        _fill2d(rows0, CHUNK, DW, 0.0)
        for t in range(rpt // CHUNK):
            pltpu.sync_copy(rows0, acc_sh.at[pl.ds(row0 + t * CHUNK, CHUNK)])
        pltpu.make_async_copy(src_hbm.at[pl.ds(0, ept)], idx_s, si).wait()
        pltpu.make_async_copy(dst_hbm.at[pl.ds(0, ept)], idx_d, si).wait()
        plsc.subcore_barrier()

        def wait_scatter(g, b):
            pltpu.make_async_copy(rows[b], acc_sh.at[idx_d.at[pl.ds(g * CHUNK, CHUNK)]],
                                  ss[b]).wait()

        def step(g, b, prefetch, drain):
            b2 = 1 - b
            if drain:
                wait_scatter(g - 1, b2)
            if prefetch:
                pltpu.async_copy(h_hbm.at[idx_s.at[pl.ds((g + 1) * CHUNK, CHUNK)]], rows[b2], sg[b2])
            pltpu.make_async_copy(h_hbm.at[idx_s.at[pl.ds(g * CHUNK, CHUNK)]], rows[b], sg[b]).wait()
            pltpu.async_copy(rows[b], acc_sh.at[idx_d.at[pl.ds(g * CHUNK, CHUNK)]], ss[b], add=True)

        pltpu.async_copy(h_hbm.at[idx_s.at[pl.ds(0, CHUNK)]], rows0, sg0)
        step(0, 0, True, False)
        step(1, 1, True, True)

        def pair(i, carry):
            g = 2 * i
            step(g, 0, True, True)
            step(g + 1, 1, True, True)
            return carry

        lax.fori_loop(1, (nchunks - 1) // 2, pair, 0)
        step(nchunks - 1, 0, False, True)
        wait_scatter(nchunks - 1, 0)

        plsc.subcore_barrier()

        pltpu.sync_copy(acc_sh.at[pl.ds(row0, rpt)],
                        part_hbm.at[c, pl.ds(row0, rpt)])

    return pl.kernel(
        body,
        out_type=jax.ShapeDtypeStruct((NC, n_pad, DW), jnp.float32),
        mesh=_sc_mesh(),
        scratch_types=[
            pltpu.VMEM((ept,), jnp.int32),
            pltpu.VMEM((ept,), jnp.int32),
            pltpu.VMEM((CHUNK, DW), jnp.float32),
            pltpu.VMEM((CHUNK, DW), jnp.float32),
            pltpu.VMEM_SHARED((n_pad, DW), jnp.float32),
            pltpu.SemaphoreType.DMA,
            pltpu.SemaphoreType.DMA,
            pltpu.SemaphoreType.DMA,
            pltpu.SemaphoreType.DMA,
            pltpu.SemaphoreType.DMA,
        ],
    )


@functools.lru_cache(maxsize=None)
def _build_deg(e, n_pad):
    ept = e // NW
    nchunks = ept // CHUNK
    rpt = n_pad // NS
    assert rpt % CHUNK == 0

    assert nchunks % 2 == 1 and nchunks >= 5

    def body(dst_hbm, degp_hbm, idx_d, ones_v, deg_sh, si, ss0, ss1):
        c = lax.axis_index("c")
        s = lax.axis_index("s")
        wid = c * NS + s
        row0 = s * rpt
        ss = (ss0, ss1)

        pltpu.async_copy(dst_hbm.at[pl.ds(wid * ept, ept)], idx_d, si)
        _fill2d(ones_v, CHUNK, DW, 0.0)
        for t in range(rpt // CHUNK):
            pltpu.sync_copy(ones_v, deg_sh.at[pl.ds(row0 + t * CHUNK, CHUNK)])
        _fill2d(ones_v, CHUNK, DW, 1.0)
        pltpu.make_async_copy(dst_hbm.at[pl.ds(0, ept)], idx_d, si).wait()
        plsc.subcore_barrier()

        def step(g, b, drain):
            if drain:
                pltpu.make_async_copy(ones_v, deg_sh.at[idx_d.at[pl.ds((g - 2) * CHUNK, CHUNK)]],
                                      ss[b]).wait()
            pltpu.async_copy(ones_v, deg_sh.at[idx_d.at[pl.ds(g * CHUNK, CHUNK)]], ss[b], add=True)

        step(0, 0, False)
        step(1, 1, False)

        def pair(i, carry):
            g = 2 * i
            step(g, 0, True)
            step(g + 1, 1, True)
            return carry

        lax.fori_loop(1, (nchunks - 1) // 2, pair, 0)
        step(nchunks - 1, 0, True)
        pltpu.make_async_copy(ones_v, deg_sh.at[idx_d.at[pl.ds((nchunks - 1) * CHUNK, CHUNK)]],
                              ss[0]).wait()
        pltpu.make_async_copy(ones_v, deg_sh.at[idx_d.at[pl.ds((nchunks - 2) * CHUNK, CHUNK)]],
                              ss[1]).wait()

        plsc.subcore_barrier()
        pltpu.sync_copy(deg_sh.at[pl.ds(row0, rpt)],
                        degp_hbm.at[c, pl.ds(row0, rpt)])

    return pl.kernel(
        body,
        out_type=jax.ShapeDtypeStruct((NC, n_pad, DW), jnp.float32),
        mesh=_sc_mesh(),
        scratch_types=[
            pltpu.VMEM((ept,), jnp.int32),
            pltpu.VMEM((CHUNK, DW), jnp.float32),
            pltpu.VMEM_SHARED((n_pad, DW), jnp.float32),
            pltpu.SemaphoreType.DMA,
            pltpu.SemaphoreType.DMA,
            pltpu.SemaphoreType.DMA,
        ],
    )


@functools.lru_cache(maxsize=None)
def _build_tc1(n_pad, h2, r_blk=512):
    def body(p_ref, dp_ref, w_ref, ha_ref, hb_ref, r_ref):
        d = dp_ref[0, :, 0:1] + dp_ref[1, :, 0:1]
        r = 1.0 / jnp.maximum(d, 1.0)
        p = p_ref[0] + p_ref[1]
        hh = jnp.dot(p, w_ref[...], preferred_element_type=jnp.float32)
        hh = jnp.maximum(hh * r, 0.0)
        ha_ref[...] = hh[:, :DW]
        hb_ref[...] = hh[:, DW:]
        r_ref[...] = r

    return pl.pallas_call(
        body,
        grid=(n_pad // r_blk,),
        in_specs=[
            pl.BlockSpec((NC, r_blk, DW), lambda i: (0, i, 0)),
            pl.BlockSpec((NC, r_blk, DW), lambda i: (0, i, 0)),
            pl.BlockSpec((DW, h2), lambda i: (0, 0)),
        ],
        out_specs=[
            pl.BlockSpec((r_blk, DW), lambda i: (i, 0)),
            pl.BlockSpec((r_blk, DW), lambda i: (i, 0)),
            pl.BlockSpec((r_blk, 1), lambda i: (i, 0)),
        ],
        out_shape=[
            jax.ShapeDtypeStruct((n_pad, DW), jnp.float32),
            jax.ShapeDtypeStruct((n_pad, DW), jnp.float32),
            jax.ShapeDtypeStruct((n_pad, 1), jnp.float32),
        ],
    )


@functools.lru_cache(maxsize=None)
def _build_tc2(n_pad, h2, r_blk=512):
    def body(pa_ref, pb_ref, r_ref, w_ref, ha_ref, hb_ref):
        p = jnp.concatenate([pa_ref[0] + pa_ref[1], pb_ref[0] + pb_ref[1]],
                            axis=1)
        hh = jnp.dot(p, w_ref[...], preferred_element_type=jnp.float32)
        hh = jnp.maximum(hh * r_ref[...], 0.0)
        ha_ref[...] = hh[:, :DW]
        hb_ref[...] = hh[:, DW:]

    return pl.pallas_call(
        body,
        grid=(n_pad // r_blk,),
        in_specs=[
            pl.BlockSpec((NC, r_blk, DW), lambda i: (0, i, 0)),
            pl.BlockSpec((NC, r_blk, DW), lambda i: (0, i, 0)),
            pl.BlockSpec((r_blk, 1), lambda i: (i, 0)),
            pl.BlockSpec((h2, h2), lambda i: (0, 0)),
        ],
        out_specs=[
            pl.BlockSpec((r_blk, DW), lambda i: (i, 0)),
            pl.BlockSpec((r_blk, DW), lambda i: (i, 0)),
        ],
        out_shape=[
            jax.ShapeDtypeStruct((n_pad, DW), jnp.float32),
            jax.ShapeDtypeStruct((n_pad, DW), jnp.float32),
        ],
    )


@functools.lru_cache(maxsize=None)
def _build_tc3(n_pad, h2, h_dim, r_blk=512):
    def body(pa_ref, pb_ref, r_ref, w_ref, h_ref):
        p = jnp.concatenate([pa_ref[0] + pa_ref[1], pb_ref[0] + pb_ref[1]],
                            axis=1)
        hh = jnp.dot(p, w_ref[...], preferred_element_type=jnp.float32)
        hh = jnp.maximum(hh * r_ref[...], 0.0)
        h_ref[...] = jnp.minimum(hh, 1000.0)

    return pl.pallas_call(
        body,
        grid=(n_pad // r_blk,),
        in_specs=[
            pl.BlockSpec((NC, r_blk, DW), lambda i: (0, i, 0)),
            pl.BlockSpec((NC, r_blk, DW), lambda i: (0, i, 0)),
            pl.BlockSpec((r_blk, 1), lambda i: (i, 0)),
            pl.BlockSpec((h2, h_dim), lambda i: (0, 0)),
        ],
        out_specs=pl.BlockSpec((r_blk, h_dim), lambda i: (i, 0)),
        out_shape=jax.ShapeDtypeStruct((n_pad, h_dim), jnp.float32),
    )


def kernel(x, edge_index, W1, W2, W3):
    n, d = x.shape
    h_dim = W1.shape[1]
    e = edge_index.shape[1]
    assert d == DW and h_dim == 192
    h2 = 2 * DW

    assert e % (NW * CHUNK) == 0
    src = edge_index[0].astype(jnp.int32)
    dst = edge_index[1].astype(jnp.int32)

    align = NS * 640
    n_pad = -(-n // align) * align

    W1p = jnp.pad(W1, ((0, 0), (0, h2 - h_dim)))
    W2p = jnp.pad(W2, ((0, h2 - h_dim), (0, h2 - h_dim)))
    W3p = jnp.pad(W3, ((0, h2 - h_dim), (0, 0)))

    agg = _build_agg(e, n_pad)

    degp = _build_deg(e, n_pad)(dst)
    part1 = agg(x, src, dst)
    h1a, h1b, rdeg = _build_tc1(n_pad, h2)(part1, degp, W1p)
    p2a = agg(h1a, src, dst)
    p2b = agg(h1b, src, dst)
    h2a, h2b = _build_tc2(n_pad, h2)(p2a, p2b, rdeg, W2p)
    p3a = agg(h2a, src, dst)
    p3b = agg(h2b, src, dst)
    h3 = _build_tc3(n_pad, h2, h_dim)(p3a, p3b, rdeg, W3p)
    return h3[:n]


# speedup vs baseline: 8.8617x; 1.1251x over previous
import functools

import jax
import jax.numpy as jnp
from jax import lax
from jax.experimental import pallas as pl
from jax.experimental.pallas import tpu as pltpu
from jax.experimental.pallas import tpu_sc as plsc

NC = 2
NS = 16
NW = NC * NS
LANES = 16
CHUNK = 80
DW = 128


def _fill2d(ref, nrows, ncols, value):
    vec = jnp.full((LANES,), value, jnp.float32)

    def row(i, carry):
        for j in range(ncols // LANES):
            ref[i, pl.ds(j * LANES, LANES)] = vec
        return carry

    lax.fori_loop(0, nrows, row, 0)


def _sc_mesh():
    return plsc.VectorSubcoreMesh(core_axis_name="c", subcore_axis_name="s")


@functools.lru_cache(maxsize=None)
def _build_agg(e, n_pad):
    ept = e // NW
    nchunks = ept // CHUNK
    rpt = n_pad // NS
    assert ept % CHUNK == 0 and rpt % CHUNK == 0

    assert nchunks % 2 == 1 and nchunks >= 5

    NB = 4
    NI = 5
    assert nchunks % 20 == 0 or (nchunks - 5) % 20 == 0
    assert nchunks >= 10

    def body(h_hbm, src_hbm, dst_hbm, part_hbm, *rest):
        rows = rest[:NB]
        isrc = rest[NB:NB + NI]
        idst = rest[NB + NI:NB + 2 * NI]
        acc_sh = rest[NB + 2 * NI]
        sems = rest[NB + 2 * NI + 1:]
        sg = sems[:NB]
        ss = sems[NB:2 * NB]
        sidx = sems[2 * NB:]
        c = lax.axis_index("c")
        s = lax.axis_index("s")
        wid = c * NS + s
        row0 = s * rpt
        e0 = wid * ept

        def load_idx(g, r):
            sl = pl.ds(pl.multiple_of(e0 + g * CHUNK, 8), CHUNK)
            pltpu.async_copy(src_hbm.at[sl], isrc[r], sidx[r])
            pltpu.async_copy(dst_hbm.at[sl], idst[r], sidx[r])

        def wait_idx(r):
            pltpu.make_async_copy(src_hbm.at[pl.ds(0, CHUNK)], isrc[r],
                                  sidx[r]).wait()
            pltpu.make_async_copy(dst_hbm.at[pl.ds(0, CHUNK)], idst[r],
                                  sidx[r]).wait()

        def gather(r, b):
            pltpu.async_copy(h_hbm.at[isrc[r]], rows[b], sg[b])

        def wait_gather(b):
            pltpu.make_async_copy(h_hbm.at[isrc[0]], rows[b], sg[b]).wait()

        def scatter(r, b):
            pltpu.async_copy(rows[b], acc_sh.at[idst[r]], ss[b], add=True)

        def wait_scatter(b, r):
            pltpu.make_async_copy(rows[b], acc_sh.at[idst[r]],
                                  ss[b]).wait()

        load_idx(0, 0)
        load_idx(1, 1)
        load_idx(2, 2)
        _fill2d(rows[0], CHUNK, DW, 0.0)
        for t in range(rpt // CHUNK):
            pltpu.sync_copy(rows[0], acc_sh.at[pl.ds(row0 + t * CHUNK, CHUNK)])
        plsc.subcore_barrier()

        def step(g, m, drain, pf_idx, pf_gather):
            if drain:
                wait_scatter((m - 2) % NB, (m - 2) % NI)
            if pf_idx:
                load_idx(g + 3, (m + 3) % NI)
            if pf_gather:
                wait_idx((m + 2) % NI)
                gather((m + 2) % NI, (m + 2) % NB)
            wait_gather(m % NB)
            scatter(m % NI, m % NB)

        wait_idx(0)
        gather(0, 0)
        wait_idx(1)
        gather(1, 1)
        step(0, 0, False, True, True)
        step(1, 1, False, True, True)

        def group(i, carry):
            for j in range(20):
                step(20 * i + 2 + j, 2 + j, True, True, True)
            return carry

        n_steady = nchunks - 5
        assert n_steady % 20 == 0
        lax.fori_loop(0, n_steady // 20, group, 0)
        step(nchunks - 3, (nchunks - 3) % 20, True, False, True)
        step(nchunks - 2, (nchunks - 2) % 20, True, False, False)
        step(nchunks - 1, (nchunks - 1) % 20, True, False, False)
        wait_scatter((nchunks - 2) % NB, (nchunks - 2) % NI)
        wait_scatter((nchunks - 1) % NB, (nchunks - 1) % NI)

        plsc.subcore_barrier()

        pltpu.sync_copy(acc_sh.at[pl.ds(row0, rpt)],
                        part_hbm.at[c, pl.ds(row0, rpt)])

    return pl.kernel(
        body,
        out_type=jax.ShapeDtypeStruct((NC, n_pad, DW), jnp.float32),
        mesh=_sc_mesh(),
        scratch_types=(
            [pltpu.VMEM((CHUNK, DW), jnp.float32)] * NB
            + [pltpu.VMEM((CHUNK,), jnp.int32)] * (2 * NI)
            + [pltpu.VMEM_SHARED((n_pad, DW), jnp.float32)]
            + [pltpu.SemaphoreType.DMA] * (2 * NB + NI)
        ),
    )


@functools.lru_cache(maxsize=None)
def _build_deg(e, n_pad):
    ept = e // NW
    nchunks = ept // CHUNK
    rpt = n_pad // NS
    assert rpt % CHUNK == 0

    assert nchunks % 2 == 1 and nchunks >= 5

    def body(dst_hbm, degp_hbm, idx_d, ones_v, deg_sh, si, ss0, ss1):
        c = lax.axis_index("c")
        s = lax.axis_index("s")
        wid = c * NS + s
        row0 = s * rpt
        ss = (ss0, ss1)

        pltpu.async_copy(dst_hbm.at[pl.ds(wid * ept, ept)], idx_d, si)
        _fill2d(ones_v, CHUNK, DW, 0.0)
        for t in range(rpt // CHUNK):
            pltpu.sync_copy(ones_v, deg_sh.at[pl.ds(row0 + t * CHUNK, CHUNK)])
        _fill2d(ones_v, CHUNK, DW, 1.0)
        pltpu.make_async_copy(dst_hbm.at[pl.ds(0, ept)], idx_d, si).wait()
        plsc.subcore_barrier()

        def step(g, b, drain):
            if drain:
                pltpu.make_async_copy(ones_v, deg_sh.at[idx_d.at[pl.ds((g - 2) * CHUNK, CHUNK)]],
                                      ss[b]).wait()
            pltpu.async_copy(ones_v, deg_sh.at[idx_d.at[pl.ds(g * CHUNK, CHUNK)]], ss[b], add=True)

        step(0, 0, False)
        step(1, 1, False)

        def pair(i, carry):
            g = 2 * i
            step(g, 0, True)
            step(g + 1, 1, True)
            return carry

        lax.fori_loop(1, (nchunks - 1) // 2, pair, 0)
        step(nchunks - 1, 0, True)
        pltpu.make_async_copy(ones_v, deg_sh.at[idx_d.at[pl.ds((nchunks - 1) * CHUNK, CHUNK)]],
                              ss[0]).wait()
        pltpu.make_async_copy(ones_v, deg_sh.at[idx_d.at[pl.ds((nchunks - 2) * CHUNK, CHUNK)]],
                              ss[1]).wait()

        plsc.subcore_barrier()
        pltpu.sync_copy(deg_sh.at[pl.ds(row0, rpt)],
                        degp_hbm.at[c, pl.ds(row0, rpt)])

    return pl.kernel(
        body,
        out_type=jax.ShapeDtypeStruct((NC, n_pad, DW), jnp.float32),
        mesh=_sc_mesh(),
        scratch_types=[
            pltpu.VMEM((ept,), jnp.int32),
            pltpu.VMEM((CHUNK, DW), jnp.float32),
            pltpu.VMEM_SHARED((n_pad, DW), jnp.float32),
            pltpu.SemaphoreType.DMA,
            pltpu.SemaphoreType.DMA,
            pltpu.SemaphoreType.DMA,
        ],
    )


@functools.lru_cache(maxsize=None)
def _build_tc1(n_pad, h2, r_blk=512):
    def body(p_ref, dp_ref, w_ref, ha_ref, hb_ref, r_ref):
        d = dp_ref[0, :, 0:1] + dp_ref[1, :, 0:1]
        r = 1.0 / jnp.maximum(d, 1.0)
        p = p_ref[0] + p_ref[1]
        hh = jnp.dot(p, w_ref[...], preferred_element_type=jnp.float32)
        hh = jnp.maximum(hh * r, 0.0)
        ha_ref[...] = hh[:, :DW]
        hb_ref[...] = hh[:, DW:]
        r_ref[...] = r

    return pl.pallas_call(
        body,
        grid=(n_pad // r_blk,),
        in_specs=[
            pl.BlockSpec((NC, r_blk, DW), lambda i: (0, i, 0)),
            pl.BlockSpec((NC, r_blk, DW), lambda i: (0, i, 0)),
            pl.BlockSpec((DW, h2), lambda i: (0, 0)),
        ],
        out_specs=[
            pl.BlockSpec((r_blk, DW), lambda i: (i, 0)),
            pl.BlockSpec((r_blk, DW), lambda i: (i, 0)),
            pl.BlockSpec((r_blk, 1), lambda i: (i, 0)),
        ],
        out_shape=[
            jax.ShapeDtypeStruct((n_pad, DW), jnp.float32),
            jax.ShapeDtypeStruct((n_pad, DW), jnp.float32),
            jax.ShapeDtypeStruct((n_pad, 1), jnp.float32),
        ],
    )


@functools.lru_cache(maxsize=None)
def _build_tc2(n_pad, h2, r_blk=512):
    def body(pa_ref, pb_ref, r_ref, w_ref, ha_ref, hb_ref):
        p = jnp.concatenate([pa_ref[0] + pa_ref[1], pb_ref[0] + pb_ref[1]],
                            axis=1)
        hh = jnp.dot(p, w_ref[...], preferred_element_type=jnp.float32)
        hh = jnp.maximum(hh * r_ref[...], 0.0)
        ha_ref[...] = hh[:, :DW]
        hb_ref[...] = hh[:, DW:]

    return pl.pallas_call(
        body,
        grid=(n_pad // r_blk,),
        in_specs=[
            pl.BlockSpec((NC, r_blk, DW), lambda i: (0, i, 0)),
            pl.BlockSpec((NC, r_blk, DW), lambda i: (0, i, 0)),
            pl.BlockSpec((r_blk, 1), lambda i: (i, 0)),
            pl.BlockSpec((h2, h2), lambda i: (0, 0)),
        ],
        out_specs=[
            pl.BlockSpec((r_blk, DW), lambda i: (i, 0)),
            pl.BlockSpec((r_blk, DW), lambda i: (i, 0)),
        ],
        out_shape=[
            jax.ShapeDtypeStruct((n_pad, DW), jnp.float32),
            jax.ShapeDtypeStruct((n_pad, DW), jnp.float32),
        ],
    )


@functools.lru_cache(maxsize=None)
def _build_tc3(n_pad, h2, h_dim, r_blk=512):
    def body(pa_ref, pb_ref, r_ref, w_ref, h_ref):
        p = jnp.concatenate([pa_ref[0] + pa_ref[1], pb_ref[0] + pb_ref[1]],
                            axis=1)
        hh = jnp.dot(p, w_ref[...], preferred_element_type=jnp.float32)
        hh = jnp.maximum(hh * r_ref[...], 0.0)
        h_ref[...] = jnp.minimum(hh, 1000.0)

    return pl.pallas_call(
        body,
        grid=(n_pad // r_blk,),
        in_specs=[
            pl.BlockSpec((NC, r_blk, DW), lambda i: (0, i, 0)),
            pl.BlockSpec((NC, r_blk, DW), lambda i: (0, i, 0)),
            pl.BlockSpec((r_blk, 1), lambda i: (i, 0)),
            pl.BlockSpec((h2, h_dim), lambda i: (0, 0)),
        ],
        out_specs=pl.BlockSpec((r_blk, h_dim), lambda i: (i, 0)),
        out_shape=jax.ShapeDtypeStruct((n_pad, h_dim), jnp.float32),
    )


def kernel(x, edge_index, W1, W2, W3):
    n, d = x.shape
    h_dim = W1.shape[1]
    e = edge_index.shape[1]
    assert d == DW and h_dim == 192
    h2 = 2 * DW

    assert e % (NW * CHUNK) == 0
    src = edge_index[0].astype(jnp.int32)
    dst = edge_index[1].astype(jnp.int32)

    align = NS * 640
    n_pad = -(-n // align) * align

    W1p = jnp.pad(W1, ((0, 0), (0, h2 - h_dim)))
    W2p = jnp.pad(W2, ((0, h2 - h_dim), (0, h2 - h_dim)))
    W3p = jnp.pad(W3, ((0, h2 - h_dim), (0, 0)))

    agg = _build_agg(e, n_pad)

    degp = _build_deg(e, n_pad)(dst)
    part1 = agg(x, src, dst)
    h1a, h1b, rdeg = _build_tc1(n_pad, h2)(part1, degp, W1p)
    p2a = agg(h1a, src, dst)
    p2b = agg(h1b, src, dst)
    h2a, h2b = _build_tc2(n_pad, h2)(p2a, p2b, rdeg, W2p)
    p3a = agg(h2a, src, dst)
    p3b = agg(h2b, src, dst)
    h3 = _build_tc3(n_pad, h2, h_dim)(p3a, p3b, rdeg, W3p)
    return h3[:n]


# speedup vs baseline: 8.8765x; 1.0017x over previous
import functools

import jax
import jax.numpy as jnp
from jax import lax
from jax.experimental import pallas as pl
from jax.experimental.pallas import tpu as pltpu
from jax.experimental.pallas import tpu_sc as plsc

NC = 2
NS = 16
NW = NC * NS
LANES = 16
CHUNK = 80
DW = 128


def _fill2d(ref, nrows, ncols, value):
    vec = jnp.full((LANES,), value, jnp.float32)

    def row(i, carry):
        for j in range(ncols // LANES):
            ref[i, pl.ds(j * LANES, LANES)] = vec
        return carry

    lax.fori_loop(0, nrows, row, 0)


def _sc_mesh():
    return plsc.VectorSubcoreMesh(core_axis_name="c", subcore_axis_name="s")


@functools.lru_cache(maxsize=None)
def _build_agg(e, n_pad):
    ept = e // NW
    nchunks = ept // CHUNK
    rpt = n_pad // NS
    assert ept % CHUNK == 0 and rpt % CHUNK == 0

    assert nchunks % 2 == 1 and nchunks >= 5

    NB = 4
    NI = 5
    assert nchunks % 20 == 0 or (nchunks - 5) % 20 == 0
    assert nchunks >= 10

    def body(h_hbm, src_hbm, dst_hbm, part_hbm, *rest):
        rows = rest[:NB]
        isrc = rest[NB:NB + NI]
        idst = rest[NB + NI:NB + 2 * NI]
        acc_sh = rest[NB + 2 * NI]
        sems = rest[NB + 2 * NI + 1:]
        sg = sems[:NB]
        ss = sems[NB:2 * NB]
        sidx = sems[2 * NB:]
        c = lax.axis_index("c")
        s = lax.axis_index("s")
        wid = c * NS + s
        row0 = s * rpt
        e0 = wid * ept

        def load_idx(g, r):
            sl = pl.ds(pl.multiple_of(e0 + g * CHUNK, 8), CHUNK)
            pltpu.async_copy(src_hbm.at[sl], isrc[r], sidx[r])
            pltpu.async_copy(dst_hbm.at[sl], idst[r], sidx[r])

        def wait_idx(r):
            pltpu.make_async_copy(src_hbm.at[pl.ds(0, CHUNK)], isrc[r],
                                  sidx[r]).wait()
            pltpu.make_async_copy(dst_hbm.at[pl.ds(0, CHUNK)], idst[r],
                                  sidx[r]).wait()

        def gather(r, b):
            pltpu.async_copy(h_hbm.at[isrc[r]], rows[b], sg[b])

        def wait_gather(b):
            pltpu.make_async_copy(h_hbm.at[isrc[0]], rows[b], sg[b]).wait()

        def scatter(r, b):
            pltpu.async_copy(rows[b], acc_sh.at[idst[r]], ss[b], add=True)

        def wait_scatter(b, r):
            pltpu.make_async_copy(rows[b], acc_sh.at[idst[r]],
                                  ss[b]).wait()

        load_idx(0, 0)
        load_idx(1, 1)
        load_idx(2, 2)
        _fill2d(rows[0], CHUNK, DW, 0.0)
        for t in range(rpt // CHUNK):
            pltpu.sync_copy(rows[0], acc_sh.at[pl.ds(row0 + t * CHUNK, CHUNK)])
        plsc.subcore_barrier()

        def step(g, m, drain, pf_idx, pf_gather):
            if drain:
                wait_scatter((m - 2) % NB, (m - 2) % NI)
            if pf_idx:
                load_idx(g + 3, (m + 3) % NI)
            if pf_gather:
                wait_idx((m + 2) % NI)
                gather((m + 2) % NI, (m + 2) % NB)
            wait_gather(m % NB)
            scatter(m % NI, m % NB)

        wait_idx(0)
        gather(0, 0)
        wait_idx(1)
        gather(1, 1)
        step(0, 0, False, True, True)
        step(1, 1, False, True, True)

        def group(i, carry):
            for j in range(20):
                step(20 * i + 2 + j, 2 + j, True, True, True)
            return carry

        n_steady = nchunks - 5
        assert n_steady % 20 == 0
        lax.fori_loop(0, n_steady // 20, group, 0)
        step(nchunks - 3, (nchunks - 3) % 20, True, False, True)
        step(nchunks - 2, (nchunks - 2) % 20, True, False, False)
        step(nchunks - 1, (nchunks - 1) % 20, True, False, False)
        wait_scatter((nchunks - 2) % NB, (nchunks - 2) % NI)
        wait_scatter((nchunks - 1) % NB, (nchunks - 1) % NI)

        plsc.subcore_barrier()

        pltpu.sync_copy(acc_sh.at[pl.ds(row0, rpt)],
                        part_hbm.at[c, pl.ds(row0, rpt)])

    return pl.kernel(
        body,
        out_type=jax.ShapeDtypeStruct((NC, n_pad, DW), jnp.float32),
        mesh=_sc_mesh(),
        scratch_types=(
            [pltpu.VMEM((CHUNK, DW), jnp.float32)] * NB
            + [pltpu.VMEM((CHUNK,), jnp.int32)] * (2 * NI)
            + [pltpu.VMEM_SHARED((n_pad, DW), jnp.float32)]
            + [pltpu.SemaphoreType.DMA] * (2 * NB + NI)
        ),
    )


@functools.lru_cache(maxsize=None)
def _build_deg(e, n_pad):
    ept = e // NW
    nchunks = ept // CHUNK
    rpt = n_pad // NS
    assert rpt % CHUNK == 0

    ND = 4
    assert (nchunks - 1) % ND == 0 and nchunks > 2 * ND

    def body(dst_hbm, degp_hbm, idx_d, ones_v, deg_sh, si, *ss):
        c = lax.axis_index("c")
        s = lax.axis_index("s")
        wid = c * NS + s
        row0 = s * rpt

        pltpu.async_copy(dst_hbm.at[pl.ds(wid * ept, ept)], idx_d, si)
        _fill2d(ones_v, CHUNK, DW, 0.0)
        for t in range(rpt // CHUNK):
            pltpu.sync_copy(ones_v, deg_sh.at[pl.ds(row0 + t * CHUNK, CHUNK)])
        _fill2d(ones_v, CHUNK, DW, 1.0)
        pltpu.make_async_copy(dst_hbm.at[pl.ds(0, ept)], idx_d, si).wait()
        plsc.subcore_barrier()

        def d_at(g):
            return deg_sh.at[idx_d.at[pl.ds(g * CHUNK, CHUNK)]]

        def step(g, b, drain):
            if drain:
                pltpu.make_async_copy(ones_v, d_at(g - ND), ss[b]).wait()
            pltpu.async_copy(ones_v, d_at(g), ss[b], add=True)

        for g in range(ND):
            step(g, g, False)

        def group(i, carry):
            g = ND * i
            for j in range(ND):
                step(g + j, j, True)
            return carry

        lax.fori_loop(1, (nchunks - 1) // ND, group, 0)
        step(nchunks - 1, (nchunks - 1) % ND, True)
        for g in range(nchunks - ND, nchunks):
            pltpu.make_async_copy(ones_v, d_at(g), ss[g % ND]).wait()

        plsc.subcore_barrier()
        pltpu.sync_copy(deg_sh.at[pl.ds(row0, rpt)],
                        degp_hbm.at[c, pl.ds(row0, rpt)])

    return pl.kernel(
        body,
        out_type=jax.ShapeDtypeStruct((NC, n_pad, DW), jnp.float32),
        mesh=_sc_mesh(),
        scratch_types=[
            pltpu.VMEM((ept,), jnp.int32),
            pltpu.VMEM((CHUNK, DW), jnp.float32),
            pltpu.VMEM_SHARED((n_pad, DW), jnp.float32),
        ] + [pltpu.SemaphoreType.DMA] * 5,
    )


@functools.lru_cache(maxsize=None)
def _build_tc1(n_pad, h2, r_blk=512):
    def body(p_ref, dp_ref, w_ref, ha_ref, hb_ref, r_ref):
        d = dp_ref[0, :, 0:1] + dp_ref[1, :, 0:1]
        r = 1.0 / jnp.maximum(d, 1.0)
        p = p_ref[0] + p_ref[1]
        hh = jnp.dot(p, w_ref[...], preferred_element_type=jnp.float32)
        hh = jnp.maximum(hh * r, 0.0)
        ha_ref[...] = hh[:, :DW]
        hb_ref[...] = hh[:, DW:]
        r_ref[...] = r

    return pl.pallas_call(
        body,
        grid=(n_pad // r_blk,),
        in_specs=[
            pl.BlockSpec((NC, r_blk, DW), lambda i: (0, i, 0)),
            pl.BlockSpec((NC, r_blk, DW), lambda i: (0, i, 0)),
            pl.BlockSpec((DW, h2), lambda i: (0, 0)),
        ],
        out_specs=[
            pl.BlockSpec((r_blk, DW), lambda i: (i, 0)),
            pl.BlockSpec((r_blk, DW), lambda i: (i, 0)),
            pl.BlockSpec((r_blk, 1), lambda i: (i, 0)),
        ],
        out_shape=[
            jax.ShapeDtypeStruct((n_pad, DW), jnp.float32),
            jax.ShapeDtypeStruct((n_pad, DW), jnp.float32),
            jax.ShapeDtypeStruct((n_pad, 1), jnp.float32),
        ],
    )


@functools.lru_cache(maxsize=None)
def _build_tc2(n_pad, h2, r_blk=512):
    def body(pa_ref, pb_ref, r_ref, w_ref, ha_ref, hb_ref):
        p = jnp.concatenate([pa_ref[0] + pa_ref[1], pb_ref[0] + pb_ref[1]],
                            axis=1)
        hh = jnp.dot(p, w_ref[...], preferred_element_type=jnp.float32)
        hh = jnp.maximum(hh * r_ref[...], 0.0)
        ha_ref[...] = hh[:, :DW]
        hb_ref[...] = hh[:, DW:]

    return pl.pallas_call(
        body,
        grid=(n_pad // r_blk,),
        in_specs=[
            pl.BlockSpec((NC, r_blk, DW), lambda i: (0, i, 0)),
            pl.BlockSpec((NC, r_blk, DW), lambda i: (0, i, 0)),
            pl.BlockSpec((r_blk, 1), lambda i: (i, 0)),
            pl.BlockSpec((h2, h2), lambda i: (0, 0)),
        ],
        out_specs=[
            pl.BlockSpec((r_blk, DW), lambda i: (i, 0)),
            pl.BlockSpec((r_blk, DW), lambda i: (i, 0)),
        ],
        out_shape=[
            jax.ShapeDtypeStruct((n_pad, DW), jnp.float32),
            jax.ShapeDtypeStruct((n_pad, DW), jnp.float32),
        ],
    )


@functools.lru_cache(maxsize=None)
def _build_tc3(n_pad, h2, h_dim, r_blk=512):
    def body(pa_ref, pb_ref, r_ref, w_ref, h_ref):
        p = jnp.concatenate([pa_ref[0] + pa_ref[1], pb_ref[0] + pb_ref[1]],
                            axis=1)
        hh = jnp.dot(p, w_ref[...], preferred_element_type=jnp.float32)
        hh = jnp.maximum(hh * r_ref[...], 0.0)
        h_ref[...] = jnp.minimum(hh, 1000.0)

    return pl.pallas_call(
        body,
        grid=(n_pad // r_blk,),
        in_specs=[
            pl.BlockSpec((NC, r_blk, DW), lambda i: (0, i, 0)),
            pl.BlockSpec((NC, r_blk, DW), lambda i: (0, i, 0)),
            pl.BlockSpec((r_blk, 1), lambda i: (i, 0)),
            pl.BlockSpec((h2, h_dim), lambda i: (0, 0)),
        ],
        out_specs=pl.BlockSpec((r_blk, h_dim), lambda i: (i, 0)),
        out_shape=jax.ShapeDtypeStruct((n_pad, h_dim), jnp.float32),
    )


def kernel(x, edge_index, W1, W2, W3):
    n, d = x.shape
    h_dim = W1.shape[1]
    e = edge_index.shape[1]
    assert d == DW and h_dim == 192
    h2 = 2 * DW

    assert e % (NW * CHUNK) == 0
    src = edge_index[0].astype(jnp.int32)
    dst = edge_index[1].astype(jnp.int32)

    align = NS * 640
    n_pad = -(-n // align) * align

    W1p = jnp.pad(W1, ((0, 0), (0, h2 - h_dim)))
    W2p = jnp.pad(W2, ((0, h2 - h_dim), (0, h2 - h_dim)))
    W3p = jnp.pad(W3, ((0, h2 - h_dim), (0, 0)))

    agg = _build_agg(e, n_pad)

    degp = _build_deg(e, n_pad)(dst)
    part1 = agg(x, src, dst)
    h1a, h1b, rdeg = _build_tc1(n_pad, h2)(part1, degp, W1p)
    p2a = agg(h1a, src, dst)
    p2b = agg(h1b, src, dst)
    h2a, h2b = _build_tc2(n_pad, h2)(p2a, p2b, rdeg, W2p)
    p3a = agg(h2a, src, dst)
    p3b = agg(h2b, src, dst)
    h3 = _build_tc3(n_pad, h2, h_dim)(p3a, p3b, rdeg, W3p)
    return h3[:n]
